# baseline (speedup 1.0000x reference)
import functools

import jax
import jax.numpy as jnp
from jax import lax
from jax.experimental import pallas as pl
from jax.experimental.pallas import tpu as pltpu
from jax.experimental.pallas import tpu_sc as plsc

N_TOKENS = 4096
D_MODEL = 1024
N_LEAF = 8
DEC_BLOCK = 1024
GM_BLOCK = 256
NB = N_TOKENS // GM_BLOCK
NSTEPS = NB + N_LEAF - 1


def _dec_body(x_ref, wbT_ref, bb_ref, leaf_ref, rank_ref, counts_ref, offs_ref, carry):
    i = pl.program_id(0)

    @pl.when(i == 0)
    def _():
        carry[...] = jnp.zeros_like(carry)

    x = x_ref[...]
    lg = jnp.dot(x, wbT_ref[...], preferred_element_type=jnp.float32)
    lg = lg + bb_ref[...]
    s = jnp.where(lg > 0, 1.0, 0.0)
    col = lax.broadcasted_iota(jnp.int32, lg.shape, 1)

    def c(k):
        return jnp.sum(jnp.where(col == k, s, 0.0), axis=1, keepdims=True)

    c0, c1, c2, c3, c4, c5, c6 = (c(k) for k in range(7))
    b0 = c0
    b1 = b0 * c2 + (1.0 - b0) * c1
    b2 = b0 * (b1 * c6 + (1.0 - b1) * c5) + (1.0 - b0) * (b1 * c4 + (1.0 - b1) * c3)
    leaf_f = 4.0 * b0 + 2.0 * b1 + b2
    leaf_ref[...] = leaf_f.astype(jnp.int32)

    f0 = ((col >> 2) & 1).astype(jnp.float32)
    f1 = ((col >> 1) & 1).astype(jnp.float32)
    f2 = (col & 1).astype(jnp.float32)
    valid = (col < N_LEAF).astype(jnp.float32)
    oh = (
        valid
        * (b0 * f0 + (1.0 - b0) * (1.0 - f0))
        * (b1 * f1 + (1.0 - b1) * (1.0 - f1))
        * (b2 * f2 + (1.0 - b2) * (1.0 - f2))
    )
    rowi = lax.broadcasted_iota(jnp.int32, (DEC_BLOCK, DEC_BLOCK), 0)
    colj = lax.broadcasted_iota(jnp.int32, (DEC_BLOCK, DEC_BLOCK), 1)
    ls = (colj < rowi).astype(jnp.float32)
    pref = jnp.dot(ls, oh, preferred_element_type=jnp.float32)
    rank = jnp.sum((pref + carry[...]) * oh, axis=1, keepdims=True)
    rank_ref[...] = rank.astype(jnp.int32)
    carry[...] = carry[...] + jnp.sum(oh, axis=0, keepdims=True)
    counts_ref[...] = carry[...].astype(jnp.int32)
    ui = lax.broadcasted_iota(jnp.int32, (128, 128), 0)
    uj = lax.broadcasted_iota(jnp.int32, (128, 128), 1)
    ut = (ui < uj).astype(jnp.float32)
    offs = jnp.dot(carry[...], ut, preferred_element_type=jnp.float32,
                   precision=lax.Precision.HIGHEST)
    offs_ref[...] = offs.astype(jnp.int32)


def _decide(xs, w_branch, b_branch):
    wbT = jnp.zeros((D_MODEL, 128), xs.dtype).at[:, :7].set(w_branch.T)
    bb = jnp.zeros((1, 128), xs.dtype).at[0, :7].set(b_branch)
    leaf, rank, counts, offs = pl.pallas_call(
        _dec_body,
        grid=(N_TOKENS // DEC_BLOCK,),
        in_specs=[
            pl.BlockSpec((DEC_BLOCK, D_MODEL), lambda i: (i, 0)),
            pl.BlockSpec((D_MODEL, 128), lambda i: (0, 0)),
            pl.BlockSpec((1, 128), lambda i: (0, 0)),
        ],
        out_specs=[
            pl.BlockSpec((DEC_BLOCK, 1), lambda i: (i, 0)),
            pl.BlockSpec((DEC_BLOCK, 1), lambda i: (i, 0)),
            pl.BlockSpec((1, 128), lambda i: (0, 0)),
            pl.BlockSpec((1, 128), lambda i: (0, 0)),
        ],
        out_shape=[
            jax.ShapeDtypeStruct((N_TOKENS, 1), jnp.int32),
            jax.ShapeDtypeStruct((N_TOKENS, 1), jnp.int32),
            jax.ShapeDtypeStruct((1, 128), jnp.int32),
            jax.ShapeDtypeStruct((1, 128), jnp.int32),
        ],
        scratch_shapes=[pltpu.VMEM((1, 128), jnp.float32)],
        compiler_params=pltpu.CompilerParams(
            dimension_semantics=("arbitrary",),
        ),
    )(xs, wbT, bb)
    return leaf.reshape(N_TOKENS), rank.reshape(N_TOKENS), counts, offs


def _make_scatter():
    info = plsc.get_sparse_core_info()
    nc, ns = info.num_cores, info.num_subcores
    nw = nc * ns
    rows_per_w = N_TOKENS // nw
    ch = 32
    n_ch = rows_per_w // ch
    mesh = plsc.VectorSubcoreMesh(core_axis_name="c", subcore_axis_name="s")

    @functools.partial(
        pl.kernel,
        mesh=mesh,
        out_type=(
            jax.ShapeDtypeStruct((N_TOKENS, D_MODEL), jnp.float32),
            jax.ShapeDtypeStruct((N_TOKENS,), jnp.int32),
        ),
        scratch_types=[
            pltpu.VMEM((2, ch), jnp.int32),
            pltpu.VMEM((2, ch, D_MODEL), jnp.float32),
            pltpu.VMEM((1, 128), jnp.int32),
            pltpu.VMEM((16,), jnp.int32),
            pltpu.VMEM((rows_per_w,), jnp.int32),
            pltpu.VMEM((rows_per_w,), jnp.int32),
            pltpu.SemaphoreType.DMA((2,)),
            pltpu.SemaphoreType.DMA((2,)),
        ],
        compiler_params=pltpu.CompilerParams(needs_layout_passes=False),
    )
    def scatter_k(xs_hbm, leaf_hbm, rank_hbm, offs_hbm, out_hbm, pos_hbm,
                  idx_v, buf, cnt_v, off_t, leaf_v, pos_v, in_sem, out_sem):
        wid = lax.axis_index("s") * nc + lax.axis_index("c")
        base = wid * rows_per_w

        def in_args(j):
            slot = j % 2
            off = base + j * ch
            return xs_hbm.at[pl.ds(off, ch)], buf.at[slot], in_sem.at[slot]

        def out_args(j):
            slot = j % 2
            return buf.at[slot], out_hbm.at[idx_v.at[slot]], out_sem.at[slot]

        pltpu.async_copy(*in_args(0))
        if n_ch > 1:
            pltpu.async_copy(*in_args(1))

        pltpu.sync_copy(offs_hbm, cnt_v)
        off_t[...] = cnt_v[0, pl.ds(0, 16)]
        pltpu.sync_copy(leaf_hbm.at[pl.ds(base, rows_per_w)], leaf_v)
        pltpu.sync_copy(rank_hbm.at[pl.ds(base, rows_per_w)], pos_v)
        for k in range(rows_per_w // 16):
            lv = leaf_v[pl.ds(16 * k, 16)]
            rv = pos_v[pl.ds(16 * k, 16)]
            pos_v[pl.ds(16 * k, 16)] = plsc.load_gather(off_t, [lv]) + rv
        pltpu.sync_copy(pos_v, pos_hbm.at[pl.ds(base, rows_per_w)])

        for j in range(n_ch):
            slot = j % 2
            for k in range(ch // 16):
                idx_v[slot, pl.ds(16 * k, 16)] = pos_v[pl.ds(j * ch + 16 * k, 16)]
            pltpu.make_async_copy(*in_args(j)).wait()
            pltpu.async_copy(*out_args(j))
            if j + 2 < n_ch:
                pltpu.make_async_copy(*out_args(j)).wait()
                pltpu.async_copy(*in_args(j + 2))
        for j in range(max(n_ch - 2, 0), n_ch):
            pltpu.make_async_copy(*out_args(j)).wait()

    return scatter_k


def _make_gather():
    info = plsc.get_sparse_core_info()
    nc, ns = info.num_cores, info.num_subcores
    nw = nc * ns
    rows_per_w = N_TOKENS // nw
    ch = 32
    n_ch = rows_per_w // ch
    mesh = plsc.VectorSubcoreMesh(core_axis_name="c", subcore_axis_name="s")

    @functools.partial(
        pl.kernel,
        mesh=mesh,
        out_type=jax.ShapeDtypeStruct((N_TOKENS, D_MODEL), jnp.float32),
        scratch_types=[
            pltpu.VMEM((rows_per_w,), jnp.int32),
            pltpu.VMEM((2, ch, D_MODEL), jnp.float32),
            pltpu.SemaphoreType.DMA((2,)),
            pltpu.SemaphoreType.DMA((2,)),
        ],
        compiler_params=pltpu.CompilerParams(needs_layout_passes=False),
    )
    def gather_k(table_hbm, idx_hbm, out_hbm, idx_v, buf, in_sem, out_sem):
        wid = lax.axis_index("s") * nc + lax.axis_index("c")
        base = wid * rows_per_w
        pltpu.sync_copy(idx_hbm.at[pl.ds(base, rows_per_w)], idx_v)

        def in_args(j):
            slot = j % 2
            return (table_hbm.at[idx_v.at[pl.ds(j * ch, ch)]], buf.at[slot],
                    in_sem.at[slot])

        def out_args(j):
            slot = j % 2
            off = base + j * ch
            return buf.at[slot], out_hbm.at[pl.ds(off, ch)], out_sem.at[slot]

        def start_in(j):
            pltpu.async_copy(*in_args(j))

        start_in(0)
        for j in range(n_ch):
            if j + 1 < n_ch:
                if j >= 1:
                    pltpu.make_async_copy(*out_args(j - 1)).wait()
                start_in(j + 1)
            pltpu.make_async_copy(*in_args(j)).wait()
            pltpu.async_copy(*out_args(j))
        for j in range(max(n_ch - 2, 0), n_ch):
            pltpu.make_async_copy(*out_args(j)).wait()

    return gather_k


_sc_cache = {}


def _sc(name):
    if name not in _sc_cache:
        _sc_cache[name] = _make_scatter() if name == "scatter" else _make_gather()
    return _sc_cache[name]


def _group_metadata(counts):
    o = jnp.concatenate([jnp.zeros((1,), jnp.int32), jnp.cumsum(counts)])
    fb = o[:-1] // GM_BLOCK
    lb = (o[1:] + GM_BLOCK - 1) // GM_BLOCK - 1
    nb = jnp.where(counts > 0, lb - fb + 1, 0)
    csteps = jnp.cumsum(nb)
    sb = csteps - nb
    total = csteps[-1]
    s_arr = jnp.arange(NSTEPS, dtype=jnp.int32)
    lid = jnp.sum((csteps[None, :] <= s_arr[:, None]).astype(jnp.int32), axis=1)
    valid = s_arr < total
    lid_c = jnp.clip(lid, 0, N_LEAF - 1)
    oh = (lid_c[:, None] == jnp.arange(N_LEAF, dtype=jnp.int32)[None, :]).astype(
        jnp.int32
    )

    def pick(tbl):
        return jnp.sum(oh * tbl[None, :], axis=1)

    bid = pick(fb) + s_arr - pick(sb)
    is_last = (s_arr == total - 1).astype(jnp.int32)
    last_lid = jnp.sum(is_last * lid_c)
    last_bid = jnp.sum(is_last * bid)
    lid_f = jnp.where(valid, lid_c, last_lid)
    bid_f = jnp.where(valid, bid, last_bid)
    startg = jnp.maximum(pick(o[:-1]), bid_f * GM_BLOCK)
    endg = jnp.minimum(pick(o[1:]), (bid_f + 1) * GM_BLOCK)
    st = jnp.where(valid, startg - bid_f * GM_BLOCK, 0)
    en = jnp.where(valid, endg - bid_f * GM_BLOCK, 0)
    return jnp.stack([bid_f, lid_f, st, en]).astype(jnp.int32)


def _step_info(counts_ref, s):
    c = [counts_ref[0, l] for l in range(N_LEAF)]
    o = [jnp.int32(0)]
    for l in range(N_LEAF):
        o.append(o[l] + c[l])
    fb = [o[l] // GM_BLOCK for l in range(N_LEAF)]
    lb = [(o[l + 1] + GM_BLOCK - 1) // GM_BLOCK - 1 for l in range(N_LEAF)]
    nb = [jnp.where(c[l] > 0, lb[l] - fb[l] + 1, 0) for l in range(N_LEAF)]
    csteps = []
    acc = jnp.int32(0)
    for l in range(N_LEAF):
        acc = acc + nb[l]
        csteps.append(acc)
    total = csteps[-1]
    sc = jnp.minimum(s, total - 1)
    lid = jnp.int32(0)
    for l in range(N_LEAF):
        lid = lid + (csteps[l] <= sc).astype(jnp.int32)
    lid = jnp.clip(lid, 0, N_LEAF - 1)

    def pick(tbl):
        v = jnp.int32(0)
        for l in range(N_LEAF):
            v = v + jnp.where(lid == l, tbl[l], 0)
        return v

    sb = pick([csteps[l] - nb[l] for l in range(N_LEAF)])
    bid = pick(fb) + sc - sb
    startg = jnp.maximum(pick(o[:-1]), bid * GM_BLOCK)
    endg = jnp.minimum(pick(o[1:]), (bid + 1) * GM_BLOCK)
    valid = s < total
    st = jnp.where(valid, startg - bid * GM_BLOCK, 0)
    en = jnp.where(valid, endg - bid * GM_BLOCK, 0)
    return bid, lid, st, en


def _gm_body(counts_ref, x_ref, W_ref, bl_ref, o_ref):
    s = pl.program_id(0)
    bid, _, st, en = _step_info(counts_ref, s)
    prev_bid, _, _, _ = _step_info(counts_ref, jnp.maximum(s - 1, 0))
    first = jnp.logical_or(s == 0, bid != prev_bid)
    y = jnp.dot(x_ref[...], W_ref[0], preferred_element_type=jnp.float32) + bl_ref[0]
    row = lax.broadcasted_iota(jnp.int32, (GM_BLOCK, 1), 0)
    m = jnp.logical_and(row >= st, row < en)
    prev = jnp.where(first, 0.0, o_ref[...])
    o_ref[...] = prev + jnp.where(m, y, 0.0)


def _grouped_matmul(xs_sorted, W_leaf, b_leaf, counts2d):
    def x_map(s, cnt):
        return _step_info(cnt, s)[0], 0

    def w_map(s, cnt):
        return _step_info(cnt, s)[1], 0, 0

    def b_map(s, cnt):
        return _step_info(cnt, s)[1], 0, 0

    grid_spec = pltpu.PrefetchScalarGridSpec(
        num_scalar_prefetch=1,
        grid=(NSTEPS,),
        in_specs=[
            pl.BlockSpec((GM_BLOCK, D_MODEL), x_map),
            pl.BlockSpec((1, D_MODEL, D_MODEL), w_map),
            pl.BlockSpec((1, 1, D_MODEL), b_map),
        ],
        out_specs=pl.BlockSpec((GM_BLOCK, D_MODEL), x_map),
    )
    return pl.pallas_call(
        _gm_body,
        grid_spec=grid_spec,
        out_shape=jax.ShapeDtypeStruct((N_TOKENS, D_MODEL), jnp.float32),
        compiler_params=pltpu.CompilerParams(
            dimension_semantics=("arbitrary",),
        ),
    )(counts2d, xs_sorted, W_leaf, b_leaf.reshape(N_LEAF, 1, D_MODEL))


def kernel(xs, w_branch, b_branch, W_leaf, b_leaf):
    leaf, rank, counts2d, offs2d = _decide(xs, w_branch, b_branch)
    xs_sorted, pos = _sc("scatter")(xs, leaf, rank, offs2d)
    out_sorted = _grouped_matmul(xs_sorted, W_leaf, b_leaf, counts2d)
    return _sc("gather")(out_sorted, pos)


# speedup vs baseline: 1.0624x; 1.0624x over previous
import functools

import jax
import jax.numpy as jnp
from jax import lax
from jax.experimental import pallas as pl
from jax.experimental.pallas import tpu as pltpu
from jax.experimental.pallas import tpu_sc as plsc

N_TOKENS = 4096
D_MODEL = 1024
N_LEAF = 8
DEC_BLOCK = 1024
GM_BLOCK = 256
NB = N_TOKENS // GM_BLOCK
NSTEPS = NB + N_LEAF - 1


def _dec_body(x_ref, wbT_ref, bb_ref, leaf_ref, rank_ref, counts_ref, offs_ref, carry):
    i = pl.program_id(0)

    @pl.when(i == 0)
    def _():
        carry[...] = jnp.zeros_like(carry)

    x = x_ref[...]
    lg = jnp.dot(x, wbT_ref[...], preferred_element_type=jnp.float32)
    lg = lg + bb_ref[...]
    s = jnp.where(lg > 0, 1.0, 0.0)
    col = lax.broadcasted_iota(jnp.int32, lg.shape, 1)

    def c(k):
        return jnp.sum(jnp.where(col == k, s, 0.0), axis=1, keepdims=True)

    c0, c1, c2, c3, c4, c5, c6 = (c(k) for k in range(7))
    b0 = c0
    b1 = b0 * c2 + (1.0 - b0) * c1
    b2 = b0 * (b1 * c6 + (1.0 - b1) * c5) + (1.0 - b0) * (b1 * c4 + (1.0 - b1) * c3)
    leaf_f = 4.0 * b0 + 2.0 * b1 + b2
    leaf_ref[...] = leaf_f.astype(jnp.int32)

    f0 = ((col >> 2) & 1).astype(jnp.float32)
    f1 = ((col >> 1) & 1).astype(jnp.float32)
    f2 = (col & 1).astype(jnp.float32)
    valid = (col < N_LEAF).astype(jnp.float32)
    oh = (
        valid
        * (b0 * f0 + (1.0 - b0) * (1.0 - f0))
        * (b1 * f1 + (1.0 - b1) * (1.0 - f1))
        * (b2 * f2 + (1.0 - b2) * (1.0 - f2))
    )
    rowi = lax.broadcasted_iota(jnp.int32, (DEC_BLOCK, DEC_BLOCK), 0)
    colj = lax.broadcasted_iota(jnp.int32, (DEC_BLOCK, DEC_BLOCK), 1)
    ls = (colj < rowi).astype(jnp.float32)
    pref = jnp.dot(ls, oh, preferred_element_type=jnp.float32)
    rank = jnp.sum((pref + carry[...]) * oh, axis=1, keepdims=True)
    rank_ref[...] = rank.astype(jnp.int32)
    carry[...] = carry[...] + jnp.sum(oh, axis=0, keepdims=True)
    counts_ref[...] = carry[...].astype(jnp.int32)
    ui = lax.broadcasted_iota(jnp.int32, (128, 128), 0)
    uj = lax.broadcasted_iota(jnp.int32, (128, 128), 1)
    ut = (ui < uj).astype(jnp.float32)
    offs = jnp.dot(carry[...], ut, preferred_element_type=jnp.float32,
                   precision=lax.Precision.HIGHEST)
    offs_ref[...] = offs.astype(jnp.int32)


def _decide(xs, w_branch, b_branch):
    wbT = jnp.zeros((D_MODEL, 128), xs.dtype).at[:, :7].set(w_branch.T)
    bb = jnp.zeros((1, 128), xs.dtype).at[0, :7].set(b_branch)
    leaf, rank, counts, offs = pl.pallas_call(
        _dec_body,
        grid=(N_TOKENS // DEC_BLOCK,),
        in_specs=[
            pl.BlockSpec((DEC_BLOCK, D_MODEL), lambda i: (i, 0)),
            pl.BlockSpec((D_MODEL, 128), lambda i: (0, 0)),
            pl.BlockSpec((1, 128), lambda i: (0, 0)),
        ],
        out_specs=[
            pl.BlockSpec((DEC_BLOCK, 1), lambda i: (i, 0)),
            pl.BlockSpec((DEC_BLOCK, 1), lambda i: (i, 0)),
            pl.BlockSpec((1, 128), lambda i: (0, 0)),
            pl.BlockSpec((1, 128), lambda i: (0, 0)),
        ],
        out_shape=[
            jax.ShapeDtypeStruct((N_TOKENS, 1), jnp.int32),
            jax.ShapeDtypeStruct((N_TOKENS, 1), jnp.int32),
            jax.ShapeDtypeStruct((1, 128), jnp.int32),
            jax.ShapeDtypeStruct((1, 128), jnp.int32),
        ],
        scratch_shapes=[pltpu.VMEM((1, 128), jnp.float32)],
        compiler_params=pltpu.CompilerParams(
            dimension_semantics=("arbitrary",),
        ),
    )(xs, wbT, bb)
    return leaf.reshape(N_TOKENS), rank.reshape(N_TOKENS), counts, offs


def _make_scatter():
    info = plsc.get_sparse_core_info()
    nc, ns = info.num_cores, info.num_subcores
    nw = nc * ns
    rows_per_w = N_TOKENS // nw
    ch = 32
    n_ch = rows_per_w // ch
    mesh = plsc.VectorSubcoreMesh(core_axis_name="c", subcore_axis_name="s")

    @functools.partial(
        pl.kernel,
        mesh=mesh,
        out_type=(
            jax.ShapeDtypeStruct((N_TOKENS, D_MODEL), jnp.float32),
            jax.ShapeDtypeStruct((N_TOKENS,), jnp.int32),
        ),
        scratch_types=[
            pltpu.VMEM((2, ch), jnp.int32),
            pltpu.VMEM((2, ch, D_MODEL), jnp.float32),
            pltpu.VMEM((1, 128), jnp.int32),
            pltpu.VMEM((16,), jnp.int32),
            pltpu.VMEM((rows_per_w,), jnp.int32),
            pltpu.VMEM((rows_per_w,), jnp.int32),
            pltpu.SemaphoreType.DMA((2,)),
            pltpu.SemaphoreType.DMA((2,)),
        ],
        compiler_params=pltpu.CompilerParams(needs_layout_passes=False),
    )
    def scatter_k(xs_hbm, leaf_hbm, rank_hbm, offs_hbm, out_hbm, pos_hbm,
                  idx_v, buf, cnt_v, off_t, leaf_v, pos_v, in_sem, out_sem):
        wid = lax.axis_index("s") * nc + lax.axis_index("c")
        base = wid * rows_per_w

        def in_args(j):
            slot = j % 2
            off = base + j * ch
            return xs_hbm.at[pl.ds(off, ch)], buf.at[slot], in_sem.at[slot]

        def out_args(j):
            slot = j % 2
            return buf.at[slot], out_hbm.at[idx_v.at[slot]], out_sem.at[slot]

        pltpu.async_copy(*in_args(0))
        if n_ch > 1:
            pltpu.async_copy(*in_args(1))

        pltpu.sync_copy(offs_hbm, cnt_v)
        off_t[...] = cnt_v[0, pl.ds(0, 16)]
        pltpu.sync_copy(leaf_hbm.at[pl.ds(base, rows_per_w)], leaf_v)
        pltpu.sync_copy(rank_hbm.at[pl.ds(base, rows_per_w)], pos_v)
        for k in range(rows_per_w // 16):
            lv = leaf_v[pl.ds(16 * k, 16)]
            rv = pos_v[pl.ds(16 * k, 16)]
            pos_v[pl.ds(16 * k, 16)] = plsc.load_gather(off_t, [lv]) + rv
        pltpu.sync_copy(pos_v, pos_hbm.at[pl.ds(base, rows_per_w)])

        for j in range(n_ch):
            slot = j % 2
            for k in range(ch // 16):
                idx_v[slot, pl.ds(16 * k, 16)] = pos_v[pl.ds(j * ch + 16 * k, 16)]
            pltpu.make_async_copy(*in_args(j)).wait()
            pltpu.async_copy(*out_args(j))
            if j + 2 < n_ch:
                pltpu.make_async_copy(*out_args(j)).wait()
                pltpu.async_copy(*in_args(j + 2))
        for j in range(max(n_ch - 2, 0), n_ch):
            pltpu.make_async_copy(*out_args(j)).wait()

    return scatter_k


def _make_gather():
    info = plsc.get_sparse_core_info()
    nc, ns = info.num_cores, info.num_subcores
    nw = nc * ns
    rows_per_w = N_TOKENS // nw
    ch = 32
    n_ch = rows_per_w // ch
    mesh = plsc.VectorSubcoreMesh(core_axis_name="c", subcore_axis_name="s")

    @functools.partial(
        pl.kernel,
        mesh=mesh,
        out_type=jax.ShapeDtypeStruct((N_TOKENS, D_MODEL), jnp.float32),
        scratch_types=[
            pltpu.VMEM((rows_per_w,), jnp.int32),
            pltpu.VMEM((2, ch, D_MODEL), jnp.float32),
            pltpu.SemaphoreType.DMA((2,)),
            pltpu.SemaphoreType.DMA((2,)),
        ],
        compiler_params=pltpu.CompilerParams(needs_layout_passes=False),
    )
    def gather_k(table_hbm, idx_hbm, out_hbm, idx_v, buf, in_sem, out_sem):
        wid = lax.axis_index("s") * nc + lax.axis_index("c")
        base = wid * rows_per_w
        pltpu.sync_copy(idx_hbm.at[pl.ds(base, rows_per_w)], idx_v)

        def in_args(j):
            slot = j % 2
            return (table_hbm.at[idx_v.at[pl.ds(j * ch, ch)]], buf.at[slot],
                    in_sem.at[slot])

        def out_args(j):
            slot = j % 2
            off = base + j * ch
            return buf.at[slot], out_hbm.at[pl.ds(off, ch)], out_sem.at[slot]

        def start_in(j):
            pltpu.async_copy(*in_args(j))

        start_in(0)
        for j in range(n_ch):
            if j + 1 < n_ch:
                if j >= 1:
                    pltpu.make_async_copy(*out_args(j - 1)).wait()
                start_in(j + 1)
            pltpu.make_async_copy(*in_args(j)).wait()
            pltpu.async_copy(*out_args(j))
        for j in range(max(n_ch - 2, 0), n_ch):
            pltpu.make_async_copy(*out_args(j)).wait()

    return gather_k


_sc_cache = {}


def _sc(name):
    if name not in _sc_cache:
        _sc_cache[name] = _make_scatter() if name == "scatter" else _make_gather()
    return _sc_cache[name]


def _group_metadata(counts):
    o = jnp.concatenate([jnp.zeros((1,), jnp.int32), jnp.cumsum(counts)])
    fb = o[:-1] // GM_BLOCK
    lb = (o[1:] + GM_BLOCK - 1) // GM_BLOCK - 1
    nb = jnp.where(counts > 0, lb - fb + 1, 0)
    csteps = jnp.cumsum(nb)
    sb = csteps - nb
    total = csteps[-1]
    s_arr = jnp.arange(NSTEPS, dtype=jnp.int32)
    lid = jnp.sum((csteps[None, :] <= s_arr[:, None]).astype(jnp.int32), axis=1)
    valid = s_arr < total
    lid_c = jnp.clip(lid, 0, N_LEAF - 1)
    oh = (lid_c[:, None] == jnp.arange(N_LEAF, dtype=jnp.int32)[None, :]).astype(
        jnp.int32
    )

    def pick(tbl):
        return jnp.sum(oh * tbl[None, :], axis=1)

    bid = pick(fb) + s_arr - pick(sb)
    is_last = (s_arr == total - 1).astype(jnp.int32)
    last_lid = jnp.sum(is_last * lid_c)
    last_bid = jnp.sum(is_last * bid)
    lid_f = jnp.where(valid, lid_c, last_lid)
    bid_f = jnp.where(valid, bid, last_bid)
    startg = jnp.maximum(pick(o[:-1]), bid_f * GM_BLOCK)
    endg = jnp.minimum(pick(o[1:]), (bid_f + 1) * GM_BLOCK)
    st = jnp.where(valid, startg - bid_f * GM_BLOCK, 0)
    en = jnp.where(valid, endg - bid_f * GM_BLOCK, 0)
    return jnp.stack([bid_f, lid_f, st, en]).astype(jnp.int32)


def _gm_body(meta_ref, x_ref, W_ref, bl_ref, o_ref):
    s = pl.program_id(0)
    st = meta_ref[2, s]
    en = meta_ref[3, s]
    bid = meta_ref[0, s]
    prev_bid = meta_ref[0, jnp.maximum(s - 1, 0)]
    first = jnp.logical_or(s == 0, bid != prev_bid)
    y = jnp.dot(x_ref[...], W_ref[0], preferred_element_type=jnp.float32) + bl_ref[0]
    row = lax.broadcasted_iota(jnp.int32, (GM_BLOCK, 1), 0)
    m = jnp.logical_and(row >= st, row < en)
    prev = jnp.where(first, 0.0, o_ref[...])
    o_ref[...] = prev + jnp.where(m, y, 0.0)


def _grouped_matmul(xs_sorted, W_leaf, b_leaf, meta):
    grid_spec = pltpu.PrefetchScalarGridSpec(
        num_scalar_prefetch=1,
        grid=(NSTEPS,),
        in_specs=[
            pl.BlockSpec((GM_BLOCK, D_MODEL), lambda s, meta: (meta[0, s], 0)),
            pl.BlockSpec((1, D_MODEL, D_MODEL), lambda s, meta: (meta[1, s], 0, 0)),
            pl.BlockSpec((1, 1, D_MODEL), lambda s, meta: (meta[1, s], 0, 0)),
        ],
        out_specs=pl.BlockSpec((GM_BLOCK, D_MODEL), lambda s, meta: (meta[0, s], 0)),
    )
    return pl.pallas_call(
        _gm_body,
        grid_spec=grid_spec,
        out_shape=jax.ShapeDtypeStruct((N_TOKENS, D_MODEL), jnp.float32),
        compiler_params=pltpu.CompilerParams(
            dimension_semantics=("arbitrary",),
        ),
    )(meta, xs_sorted, W_leaf, b_leaf.reshape(N_LEAF, 1, D_MODEL))


def kernel(xs, w_branch, b_branch, W_leaf, b_leaf):
    leaf, rank, counts2d, offs2d = _decide(xs, w_branch, b_branch)
    meta = _group_metadata(counts2d[0, :N_LEAF])
    xs_sorted, pos = _sc("scatter")(xs, leaf, rank, offs2d)
    out_sorted = _grouped_matmul(xs_sorted, W_leaf, b_leaf, meta)
    return _sc("gather")(out_sorted, pos)


# speedup vs baseline: 1.0996x; 1.0350x over previous
import functools

import jax
import jax.numpy as jnp
from jax import lax
from jax.experimental import pallas as pl
from jax.experimental.pallas import tpu as pltpu
from jax.experimental.pallas import tpu_sc as plsc

N_TOKENS = 4096
D_MODEL = 1024
N_LEAF = 8
DEC_BLOCK = 1024
GM_BLOCK = 512
NB = N_TOKENS // GM_BLOCK
NSTEPS = NB + N_LEAF - 1


def _dec_body(x_ref, wbT_ref, bb_ref, leaf_ref, rank_ref, counts_ref, offs_ref, carry):
    i = pl.program_id(0)

    @pl.when(i == 0)
    def _():
        carry[...] = jnp.zeros_like(carry)

    x = x_ref[...]
    lg = jnp.dot(x, wbT_ref[...], preferred_element_type=jnp.float32)
    lg = lg + bb_ref[...]
    s = jnp.where(lg > 0, 1.0, 0.0)
    col = lax.broadcasted_iota(jnp.int32, lg.shape, 1)

    def c(k):
        return jnp.sum(jnp.where(col == k, s, 0.0), axis=1, keepdims=True)

    c0, c1, c2, c3, c4, c5, c6 = (c(k) for k in range(7))
    b0 = c0
    b1 = b0 * c2 + (1.0 - b0) * c1
    b2 = b0 * (b1 * c6 + (1.0 - b1) * c5) + (1.0 - b0) * (b1 * c4 + (1.0 - b1) * c3)
    leaf_f = 4.0 * b0 + 2.0 * b1 + b2
    leaf_ref[...] = leaf_f.astype(jnp.int32)

    f0 = ((col >> 2) & 1).astype(jnp.float32)
    f1 = ((col >> 1) & 1).astype(jnp.float32)
    f2 = (col & 1).astype(jnp.float32)
    valid = (col < N_LEAF).astype(jnp.float32)
    oh = (
        valid
        * (b0 * f0 + (1.0 - b0) * (1.0 - f0))
        * (b1 * f1 + (1.0 - b1) * (1.0 - f1))
        * (b2 * f2 + (1.0 - b2) * (1.0 - f2))
    )
    rowi = lax.broadcasted_iota(jnp.int32, (DEC_BLOCK, DEC_BLOCK), 0)
    colj = lax.broadcasted_iota(jnp.int32, (DEC_BLOCK, DEC_BLOCK), 1)
    ls = (colj < rowi).astype(jnp.float32)
    pref = jnp.dot(ls, oh, preferred_element_type=jnp.float32)
    rank = jnp.sum((pref + carry[...]) * oh, axis=1, keepdims=True)
    rank_ref[...] = rank.astype(jnp.int32)
    carry[...] = carry[...] + jnp.sum(oh, axis=0, keepdims=True)
    counts_ref[...] = carry[...].astype(jnp.int32)
    ui = lax.broadcasted_iota(jnp.int32, (128, 128), 0)
    uj = lax.broadcasted_iota(jnp.int32, (128, 128), 1)
    ut = (ui < uj).astype(jnp.float32)
    offs = jnp.dot(carry[...], ut, preferred_element_type=jnp.float32,
                   precision=lax.Precision.HIGHEST)
    offs_ref[...] = offs.astype(jnp.int32)


def _decide(xs, w_branch, b_branch):
    wbT = jnp.zeros((D_MODEL, 128), xs.dtype).at[:, :7].set(w_branch.T)
    bb = jnp.zeros((1, 128), xs.dtype).at[0, :7].set(b_branch)
    leaf, rank, counts, offs = pl.pallas_call(
        _dec_body,
        grid=(N_TOKENS // DEC_BLOCK,),
        in_specs=[
            pl.BlockSpec((DEC_BLOCK, D_MODEL), lambda i: (i, 0)),
            pl.BlockSpec((D_MODEL, 128), lambda i: (0, 0)),
            pl.BlockSpec((1, 128), lambda i: (0, 0)),
        ],
        out_specs=[
            pl.BlockSpec((DEC_BLOCK, 1), lambda i: (i, 0)),
            pl.BlockSpec((DEC_BLOCK, 1), lambda i: (i, 0)),
            pl.BlockSpec((1, 128), lambda i: (0, 0)),
            pl.BlockSpec((1, 128), lambda i: (0, 0)),
        ],
        out_shape=[
            jax.ShapeDtypeStruct((N_TOKENS, 1), jnp.int32),
            jax.ShapeDtypeStruct((N_TOKENS, 1), jnp.int32),
            jax.ShapeDtypeStruct((1, 128), jnp.int32),
            jax.ShapeDtypeStruct((1, 128), jnp.int32),
        ],
        scratch_shapes=[pltpu.VMEM((1, 128), jnp.float32)],
        compiler_params=pltpu.CompilerParams(
            dimension_semantics=("arbitrary",),
        ),
    )(xs, wbT, bb)
    return leaf.reshape(N_TOKENS), rank.reshape(N_TOKENS), counts, offs


def _make_scatter():
    info = plsc.get_sparse_core_info()
    nc, ns = info.num_cores, info.num_subcores
    nw = nc * ns
    rows_per_w = N_TOKENS // nw
    ch = 32
    n_ch = rows_per_w // ch
    mesh = plsc.VectorSubcoreMesh(core_axis_name="c", subcore_axis_name="s")

    @functools.partial(
        pl.kernel,
        mesh=mesh,
        out_type=(
            jax.ShapeDtypeStruct((N_TOKENS, D_MODEL), jnp.float32),
            jax.ShapeDtypeStruct((N_TOKENS,), jnp.int32),
        ),
        scratch_types=[
            pltpu.VMEM((2, ch), jnp.int32),
            pltpu.VMEM((2, ch, D_MODEL), jnp.float32),
            pltpu.VMEM((1, 128), jnp.int32),
            pltpu.VMEM((16,), jnp.int32),
            pltpu.VMEM((rows_per_w,), jnp.int32),
            pltpu.VMEM((rows_per_w,), jnp.int32),
            pltpu.SemaphoreType.DMA((2,)),
            pltpu.SemaphoreType.DMA((2,)),
        ],
        compiler_params=pltpu.CompilerParams(needs_layout_passes=False),
    )
    def scatter_k(xs_hbm, leaf_hbm, rank_hbm, offs_hbm, out_hbm, pos_hbm,
                  idx_v, buf, cnt_v, off_t, leaf_v, pos_v, in_sem, out_sem):
        wid = lax.axis_index("s") * nc + lax.axis_index("c")
        base = wid * rows_per_w

        def in_args(j):
            slot = j % 2
            off = base + j * ch
            return xs_hbm.at[pl.ds(off, ch)], buf.at[slot], in_sem.at[slot]

        def out_args(j):
            slot = j % 2
            return buf.at[slot], out_hbm.at[idx_v.at[slot]], out_sem.at[slot]

        pltpu.async_copy(*in_args(0))
        if n_ch > 1:
            pltpu.async_copy(*in_args(1))

        pltpu.sync_copy(offs_hbm, cnt_v)
        off_t[...] = cnt_v[0, pl.ds(0, 16)]
        pltpu.sync_copy(leaf_hbm.at[pl.ds(base, rows_per_w)], leaf_v)
        pltpu.sync_copy(rank_hbm.at[pl.ds(base, rows_per_w)], pos_v)
        for k in range(rows_per_w // 16):
            lv = leaf_v[pl.ds(16 * k, 16)]
            rv = pos_v[pl.ds(16 * k, 16)]
            pos_v[pl.ds(16 * k, 16)] = plsc.load_gather(off_t, [lv]) + rv
        pltpu.sync_copy(pos_v, pos_hbm.at[pl.ds(base, rows_per_w)])

        for j in range(n_ch):
            slot = j % 2
            for k in range(ch // 16):
                idx_v[slot, pl.ds(16 * k, 16)] = pos_v[pl.ds(j * ch + 16 * k, 16)]
            pltpu.make_async_copy(*in_args(j)).wait()
            pltpu.async_copy(*out_args(j))
            if j + 2 < n_ch:
                pltpu.make_async_copy(*out_args(j)).wait()
                pltpu.async_copy(*in_args(j + 2))
        for j in range(max(n_ch - 2, 0), n_ch):
            pltpu.make_async_copy(*out_args(j)).wait()

    return scatter_k


def _make_gather():
    info = plsc.get_sparse_core_info()
    nc, ns = info.num_cores, info.num_subcores
    nw = nc * ns
    rows_per_w = N_TOKENS // nw
    ch = 32
    n_ch = rows_per_w // ch
    mesh = plsc.VectorSubcoreMesh(core_axis_name="c", subcore_axis_name="s")

    @functools.partial(
        pl.kernel,
        mesh=mesh,
        out_type=jax.ShapeDtypeStruct((N_TOKENS, D_MODEL), jnp.float32),
        scratch_types=[
            pltpu.VMEM((rows_per_w,), jnp.int32),
            pltpu.VMEM((2, ch, D_MODEL), jnp.float32),
            pltpu.SemaphoreType.DMA((2,)),
            pltpu.SemaphoreType.DMA((2,)),
        ],
        compiler_params=pltpu.CompilerParams(needs_layout_passes=False),
    )
    def gather_k(table_hbm, idx_hbm, out_hbm, idx_v, buf, in_sem, out_sem):
        wid = lax.axis_index("s") * nc + lax.axis_index("c")
        base = wid * rows_per_w
        pltpu.sync_copy(idx_hbm.at[pl.ds(base, rows_per_w)], idx_v)

        def in_args(j):
            slot = j % 2
            return (table_hbm.at[idx_v.at[pl.ds(j * ch, ch)]], buf.at[slot],
                    in_sem.at[slot])

        def out_args(j):
            slot = j % 2
            off = base + j * ch
            return buf.at[slot], out_hbm.at[pl.ds(off, ch)], out_sem.at[slot]

        def start_in(j):
            pltpu.async_copy(*in_args(j))

        start_in(0)
        for j in range(n_ch):
            if j + 1 < n_ch:
                if j >= 1:
                    pltpu.make_async_copy(*out_args(j - 1)).wait()
                start_in(j + 1)
            pltpu.make_async_copy(*in_args(j)).wait()
            pltpu.async_copy(*out_args(j))
        for j in range(max(n_ch - 2, 0), n_ch):
            pltpu.make_async_copy(*out_args(j)).wait()

    return gather_k


_sc_cache = {}


def _sc(name):
    if name not in _sc_cache:
        _sc_cache[name] = _make_scatter() if name == "scatter" else _make_gather()
    return _sc_cache[name]


def _group_metadata(counts):
    o = jnp.concatenate([jnp.zeros((1,), jnp.int32), jnp.cumsum(counts)])
    fb = o[:-1] // GM_BLOCK
    lb = (o[1:] + GM_BLOCK - 1) // GM_BLOCK - 1
    nb = jnp.where(counts > 0, lb - fb + 1, 0)
    csteps = jnp.cumsum(nb)
    sb = csteps - nb
    total = csteps[-1]
    s_arr = jnp.arange(NSTEPS, dtype=jnp.int32)
    lid = jnp.sum((csteps[None, :] <= s_arr[:, None]).astype(jnp.int32), axis=1)
    valid = s_arr < total
    lid_c = jnp.clip(lid, 0, N_LEAF - 1)
    oh = (lid_c[:, None] == jnp.arange(N_LEAF, dtype=jnp.int32)[None, :]).astype(
        jnp.int32
    )

    def pick(tbl):
        return jnp.sum(oh * tbl[None, :], axis=1)

    bid = pick(fb) + s_arr - pick(sb)
    is_last = (s_arr == total - 1).astype(jnp.int32)
    last_lid = jnp.sum(is_last * lid_c)
    last_bid = jnp.sum(is_last * bid)
    lid_f = jnp.where(valid, lid_c, last_lid)
    bid_f = jnp.where(valid, bid, last_bid)
    startg = jnp.maximum(pick(o[:-1]), bid_f * GM_BLOCK)
    endg = jnp.minimum(pick(o[1:]), (bid_f + 1) * GM_BLOCK)
    st = jnp.where(valid, startg - bid_f * GM_BLOCK, 0)
    en = jnp.where(valid, endg - bid_f * GM_BLOCK, 0)
    return jnp.stack([bid_f, lid_f, st, en]).astype(jnp.int32)


def _gm_body(meta_ref, x_ref, W_ref, bl_ref, o_ref):
    s = pl.program_id(0)
    st = meta_ref[2, s]
    en = meta_ref[3, s]
    bid = meta_ref[0, s]
    prev_bid = meta_ref[0, jnp.maximum(s - 1, 0)]
    first = jnp.logical_or(s == 0, bid != prev_bid)
    y = jnp.dot(x_ref[...], W_ref[0], preferred_element_type=jnp.float32) + bl_ref[0]
    row = lax.broadcasted_iota(jnp.int32, (GM_BLOCK, 1), 0)
    m = jnp.logical_and(row >= st, row < en)
    prev = jnp.where(first, 0.0, o_ref[...])
    o_ref[...] = prev + jnp.where(m, y, 0.0)


def _grouped_matmul(xs_sorted, W_leaf, b_leaf, meta):
    grid_spec = pltpu.PrefetchScalarGridSpec(
        num_scalar_prefetch=1,
        grid=(NSTEPS,),
        in_specs=[
            pl.BlockSpec((GM_BLOCK, D_MODEL), lambda s, meta: (meta[0, s], 0)),
            pl.BlockSpec((1, D_MODEL, D_MODEL), lambda s, meta: (meta[1, s], 0, 0)),
            pl.BlockSpec((1, 1, D_MODEL), lambda s, meta: (meta[1, s], 0, 0)),
        ],
        out_specs=pl.BlockSpec((GM_BLOCK, D_MODEL), lambda s, meta: (meta[0, s], 0)),
    )
    return pl.pallas_call(
        _gm_body,
        grid_spec=grid_spec,
        out_shape=jax.ShapeDtypeStruct((N_TOKENS, D_MODEL), jnp.float32),
        compiler_params=pltpu.CompilerParams(
            dimension_semantics=("arbitrary",),
        ),
    )(meta, xs_sorted, W_leaf, b_leaf.reshape(N_LEAF, 1, D_MODEL))


def kernel(xs, w_branch, b_branch, W_leaf, b_leaf):
    leaf, rank, counts2d, offs2d = _decide(xs, w_branch, b_branch)
    meta = _group_metadata(counts2d[0, :N_LEAF])
    xs_sorted, pos = _sc("scatter")(xs, leaf, rank, offs2d)
    out_sorted = _grouped_matmul(xs_sorted, W_leaf, b_leaf, meta)
    return _sc("gather")(out_sorted, pos)
